# Initial kernel scaffold; baseline (speedup 1.0000x reference)
import functools
import math

import jax
import jax.numpy as jnp
import numpy as np
from jax.experimental import pallas as pl
from jax.experimental.pallas import tpu as pltpu

_T = 200
_B = 64
_F = 64
_D_OB = 4
_D_PE = 16
_N_HEADS = 16
_D_INNER = 1024
_N_LAYERS = 2
_D_STATIC = 9
_D_TR = 256 + _D_PE
_IN_CH = _T * _D_OB
_D_FINAL = _D_TR + _F
_DH = _D_TR // _N_HEADS
_DHP = 32
_RB = 512


def _gnn1_body(x_ref, rep_ref, rtile_ref, w1t_ref, b1_ref, out_ref):
    xrep = jnp.dot(x_ref[...], rep_ref[...], preferred_element_type=jnp.float32)
    sd = jnp.maximum(xrep * rtile_ref[...], 0.0)
    z = jnp.dot(sd, w1t_ref[...], preferred_element_type=jnp.float32) + b1_ref[...]
    z = jnp.maximum(z, 0.0)
    z3 = z.reshape(_RB // _F, _F, _IN_CH)
    out_ref[...] = jnp.mean(z3, axis=1)


def _gnn2_body(v1_ref, w2t_ref, b2_ref, out_ref):
    z = jnp.dot(v1_ref[...], w2t_ref[...], preferred_element_type=jnp.float32)
    out_ref[...] = jnp.maximum(z + b2_ref[...], 0.0)


def _ln_in(x, g, b):
    mu = jnp.mean(x, axis=-1, keepdims=True)
    d = x - mu
    v = jnp.mean(d * d, axis=-1, keepdims=True)
    return d / jnp.sqrt(v + 1e-5) * g + b


def _tr_body(u_ref, ts_ref, len_ref, st_ref, tile_ref, itau_ref,
             wqkv_ref, bqkv_ref, wo_ref, bo_ref,
             g1_ref, c1_ref, g2_ref, c2_ref,
             w1_ref, b1_ref, w2_ref, b2_ref,
             stw_ref, stb_ref, m1w_ref, m1b_ref, m2w_ref, m2b_ref,
             out_ref):
    lenf = len_ref[0]
    ts = ts_ref[0]
    scaled = ts * itau_ref[...]
    pe = jnp.concatenate([jnp.sin(scaled), jnp.cos(scaled)], axis=1)
    u = u_ref[0]
    x256 = jnp.dot(u, tile_ref[...], preferred_element_type=jnp.float32)
    x = jnp.concatenate([x256, pe], axis=1)

    col = jax.lax.broadcasted_iota(jnp.float32, (_T, _T), 1)
    neg = jnp.float32(-1e9)
    scale = jnp.float32(1.0 / math.sqrt(_DH))

    for l in range(_N_LAYERS):
        qkv = jnp.dot(x, wqkv_ref[l], preferred_element_type=jnp.float32)
        qkv = qkv + bqkv_ref[l]
        outs = []
        for h in range(_N_HEADS):
            q = jax.lax.slice(qkv, (0, _DHP * h), (_T, _DHP * h + _DHP))
            k = jax.lax.slice(qkv, (0, 512 + _DHP * h), (_T, 512 + _DHP * h + _DHP))
            v = jax.lax.slice(qkv, (0, 1024 + _DHP * h), (_T, 1024 + _DHP * h + _DHP))
            s = jax.lax.dot_general(q, k, (((1,), (1,)), ((), ())),
                                    preferred_element_type=jnp.float32) * scale
            s = jnp.where(col >= lenf, neg, s)
            m = jnp.max(s, axis=1, keepdims=True)
            e = jnp.exp(s - m)
            p = e / jnp.sum(e, axis=1, keepdims=True)
            outs.append(jnp.dot(p, v, preferred_element_type=jnp.float32))
        ao = jnp.concatenate(outs, axis=1)
        y = jnp.dot(ao, wo_ref[l], preferred_element_type=jnp.float32)
        x = _ln_in(y + bo_ref[l] + x, g1_ref[l], c1_ref[l])
        ff = jnp.dot(x, w1_ref[l], preferred_element_type=jnp.float32) + b1_ref[l]
        ff = jnp.maximum(ff, 0.0)
        ff = jnp.dot(ff, w2_ref[l], preferred_element_type=jnp.float32) + b2_ref[l]
        x = _ln_in(x + ff, g2_ref[l], c2_ref[l])

    trow = jax.lax.broadcasted_iota(jnp.float32, (_T, 1), 0)
    w = jnp.where(trow < lenf, 1.0, 0.0)
    agg = jnp.sum(x * w, axis=0, keepdims=True) / (lenf + 1.0)
    emb = jnp.dot(st_ref[0], stw_ref[...], preferred_element_type=jnp.float32)
    emb = emb + stb_ref[...]
    o = jnp.concatenate([agg, emb], axis=1)
    hh = jnp.dot(o, m1w_ref[...], preferred_element_type=jnp.float32) + m1b_ref[...]
    hh = jnp.maximum(hh, 0.0)
    logits = jnp.dot(hh, m2w_ref[...], preferred_element_type=jnp.float32) + m2b_ref[...]
    m = jnp.max(logits, axis=1, keepdims=True)
    e = jnp.exp(logits - m)
    out_ref[0] = e / jnp.sum(e, axis=1, keepdims=True)


def _const_spec(shape):
    nd = len(shape)
    return pl.BlockSpec(shape, lambda *_: (0,) * nd)


@jax.jit
def kernel(X, timestamps, lengths, missing_mask, static, label, R_u, static_W,
           static_b, gnn1_Wv, gnn1_bv, gnn2_Wv, gnn2_bv, tr_Wqkv, tr_bqkv,
           tr_Wo, tr_bo, tr_ln1_g, tr_ln1_b, tr_ln2_g, tr_ln2_b, tr_W1, tr_b1,
           tr_W2, tr_b2, mlp_W1, mlp_b1, mlp_W2, mlp_b2):
    f32 = jnp.float32

    Xt = X.transpose(1, 2, 0).reshape(_B * _F, _T)
    rep = jnp.asarray(np.repeat(np.eye(_T, dtype=np.float32), _D_OB, axis=1))
    Ru2 = R_u.reshape(_F, _D_OB)
    rtile = jnp.broadcast_to(Ru2[None, :, None, :], (_RB // _F, _F, _T, _D_OB))
    rtile = rtile.reshape(_RB, _IN_CH)
    w1t = gnn1_Wv.T
    b1v = gnn1_bv.reshape(1, _IN_CH)

    v1 = pl.pallas_call(
        _gnn1_body,
        grid=(_B * _F // _RB,),
        in_specs=[
            pl.BlockSpec((_RB, _T), lambda i: (i, 0)),
            _const_spec((_T, _IN_CH)),
            _const_spec((_RB, _IN_CH)),
            _const_spec((_IN_CH, _IN_CH)),
            _const_spec((1, _IN_CH)),
        ],
        out_specs=pl.BlockSpec((_RB // _F, _IN_CH), lambda i: (i, 0)),
        out_shape=jax.ShapeDtypeStruct((_B, _IN_CH), f32),
    )(Xt, rep, rtile, w1t, b1v)

    v2 = pl.pallas_call(
        _gnn2_body,
        in_specs=[
            pl.BlockSpec((_B, _IN_CH), lambda: (0, 0)),
            pl.BlockSpec((_IN_CH, _IN_CH), lambda: (0, 0)),
            pl.BlockSpec((1, _IN_CH), lambda: (0, 0)),
        ],
        out_specs=pl.BlockSpec((_B, _IN_CH), lambda: (0, 0)),
        out_shape=jax.ShapeDtypeStruct((_B, _IN_CH), f32),
    )(v1, gnn2_Wv.T, gnn2_bv.reshape(1, _IN_CH))

    u = v2.reshape(_B, _T, _D_OB)

    wq = tr_Wqkv.reshape(_N_LAYERS, 3, _N_HEADS, _DH, _D_TR)
    wq = jnp.pad(wq, ((0, 0), (0, 0), (0, 0), (0, _DHP - _DH), (0, 0)))
    wqkv_p = wq.transpose(0, 4, 1, 2, 3).reshape(_N_LAYERS, _D_TR, 3 * _N_HEADS * _DHP)
    bq = tr_bqkv.reshape(_N_LAYERS, 3, _N_HEADS, _DH)
    bq = jnp.pad(bq, ((0, 0), (0, 0), (0, 0), (0, _DHP - _DH)))
    bqkv_p = bq.reshape(_N_LAYERS, 1, 3 * _N_HEADS * _DHP)
    wo = tr_Wo.transpose(0, 2, 1).reshape(_N_LAYERS, _N_HEADS, _DH, _D_TR)
    wo_p = jnp.pad(wo, ((0, 0), (0, 0), (0, _DHP - _DH), (0, 0)))
    wo_p = wo_p.reshape(_N_LAYERS, _N_HEADS * _DHP, _D_TR)

    tile = jnp.asarray(np.tile(np.eye(_D_OB, dtype=np.float32), (1, _F)))
    itau = jnp.asarray(
        (1.0 / (float(_T) ** np.linspace(0.0, 1.0, _D_PE // 2))).astype(np.float32)
    ).reshape(1, _D_PE // 2)
    tsr = timestamps.T.reshape(_B, _T, 1)
    lenf = lengths.astype(f32).reshape(_B, 1, 1)
    NH3 = 3 * _N_HEADS * _DHP

    def lspec(s2, s3):
        return pl.BlockSpec((1, s2, s3), lambda b: (b, 0, 0))

    def wl(s2, s3):
        return pl.BlockSpec((_N_LAYERS, s2, s3), lambda b: (0, 0, 0))

    probs = pl.pallas_call(
        _tr_body,
        grid=(_B,),
        in_specs=[
            lspec(_T, _D_OB),
            lspec(_T, 1),
            lspec(1, 1),
            lspec(1, _D_STATIC),
            _const_spec((_D_OB, 256)),
            _const_spec((1, _D_PE // 2)),
            wl(_D_TR, NH3),
            wl(1, NH3),
            wl(_N_HEADS * _DHP, _D_TR),
            wl(1, _D_TR),
            wl(1, _D_TR), wl(1, _D_TR),
            wl(1, _D_TR), wl(1, _D_TR),
            wl(_D_TR, _D_INNER),
            wl(1, _D_INNER),
            wl(_D_INNER, _D_TR),
            wl(1, _D_TR),
            _const_spec((_D_STATIC, _F)),
            _const_spec((1, _F)),
            _const_spec((_D_FINAL, _D_FINAL)),
            _const_spec((1, _D_FINAL)),
            _const_spec((_D_FINAL, 2)),
            _const_spec((1, 2)),
        ],
        out_specs=pl.BlockSpec((1, 1, 2), lambda b: (b, 0, 0)),
        out_shape=jax.ShapeDtypeStruct((_B, 1, 2), f32),
    )(
        u, tsr, lenf, static.reshape(_B, 1, _D_STATIC), tile, itau,
        wqkv_p, bqkv_p, wo_p, tr_bo.reshape(_N_LAYERS, 1, _D_TR),
        tr_ln1_g.reshape(_N_LAYERS, 1, _D_TR), tr_ln1_b.reshape(_N_LAYERS, 1, _D_TR),
        tr_ln2_g.reshape(_N_LAYERS, 1, _D_TR), tr_ln2_b.reshape(_N_LAYERS, 1, _D_TR),
        tr_W1.transpose(0, 2, 1), tr_b1.reshape(_N_LAYERS, 1, _D_INNER),
        tr_W2.transpose(0, 2, 1), tr_b2.reshape(_N_LAYERS, 1, _D_TR),
        static_W.T, static_b.reshape(1, _F),
        mlp_W1.T, mlp_b1.reshape(1, _D_FINAL),
        mlp_W2.T, mlp_b2.reshape(1, 2),
    )
    return probs.reshape(_B, 2)


# speedup vs baseline: 7.1079x; 7.1079x over previous
import functools
import math

import jax
import jax.numpy as jnp
import numpy as np
from jax.experimental import pallas as pl
from jax.experimental.pallas import tpu as pltpu

_T = 200
_B = 64
_F = 64
_D_OB = 4
_D_PE = 16
_N_HEADS = 16
_D_INNER = 1024
_N_LAYERS = 2
_D_STATIC = 9
_D_TR = 256 + _D_PE
_IN_CH = _T * _D_OB
_D_FINAL = _D_TR + _F
_DH = _D_TR // _N_HEADS
_DHP = 32
_RB = 512


def _gnn1_body(x_ref, rep_ref, rtile_ref, w1t_ref, b1_ref, out_ref):
    xrep = jnp.dot(x_ref[...], rep_ref[...], preferred_element_type=jnp.float32)
    sd = jnp.maximum(xrep * rtile_ref[...], 0.0)
    z = jnp.dot(sd, w1t_ref[...], preferred_element_type=jnp.float32) + b1_ref[...]
    z = jnp.maximum(z, 0.0)
    z3 = z.reshape(_RB // _F, _F, _IN_CH)
    out_ref[...] = jnp.mean(z3, axis=1)


def _gnn2_body(v1_ref, w2t_ref, b2_ref, out_ref):
    z = jnp.dot(v1_ref[...], w2t_ref[...], preferred_element_type=jnp.float32)
    out_ref[...] = jnp.maximum(z + b2_ref[...], 0.0)


def _ln_in(x, g, b):
    mu = jnp.mean(x, axis=-1, keepdims=True)
    d = x - mu
    v = jnp.mean(d * d, axis=-1, keepdims=True)
    return d / jnp.sqrt(v + 1e-5) * g + b


def _tr_body(u_ref, ts_ref, len_ref, st_ref, tile_ref, itau_ref,
             wqkv_ref, bqkv_ref, wo_ref, bo_ref,
             g1_ref, c1_ref, g2_ref, c2_ref,
             w1_ref, b1_ref, w2_ref, b2_ref,
             stw_ref, stb_ref, m1w_ref, m1b_ref, m2w_ref, m2b_ref,
             out_ref):
    lenf = len_ref[0]
    ts = ts_ref[0]
    scaled = ts * itau_ref[...]
    pe = jnp.concatenate([jnp.sin(scaled), jnp.cos(scaled)], axis=1)
    u = u_ref[0]
    x256 = jnp.dot(u, tile_ref[...], preferred_element_type=jnp.float32)
    x = jnp.concatenate([x256, pe], axis=1)

    col = jax.lax.broadcasted_iota(jnp.int32, (_T, _T), 1).astype(jnp.float32)
    neg = jnp.float32(-1e9)
    scale = jnp.float32(1.0 / math.sqrt(_DH))

    for l in range(_N_LAYERS):
        qkv = jnp.dot(x, wqkv_ref[l], preferred_element_type=jnp.float32)
        qkv = qkv + bqkv_ref[l]
        outs = []
        for h in range(_N_HEADS):
            q = jax.lax.slice(qkv, (0, _DHP * h), (_T, _DHP * h + _DHP))
            k = jax.lax.slice(qkv, (0, 512 + _DHP * h), (_T, 512 + _DHP * h + _DHP))
            v = jax.lax.slice(qkv, (0, 1024 + _DHP * h), (_T, 1024 + _DHP * h + _DHP))
            s = jax.lax.dot_general(q, k, (((1,), (1,)), ((), ())),
                                    preferred_element_type=jnp.float32) * scale
            s = jnp.where(col >= lenf, neg, s)
            m = jnp.max(s, axis=1, keepdims=True)
            e = jnp.exp(s - m)
            p = e / jnp.sum(e, axis=1, keepdims=True)
            outs.append(jnp.dot(p, v, preferred_element_type=jnp.float32))
        ao = jnp.concatenate(outs, axis=1)
        y = jnp.dot(ao, wo_ref[l], preferred_element_type=jnp.float32)
        x = _ln_in(y + bo_ref[l] + x, g1_ref[l], c1_ref[l])
        ff = jnp.dot(x, w1_ref[l], preferred_element_type=jnp.float32) + b1_ref[l]
        ff = jnp.maximum(ff, 0.0)
        ff = jnp.dot(ff, w2_ref[l], preferred_element_type=jnp.float32) + b2_ref[l]
        x = _ln_in(x + ff, g2_ref[l], c2_ref[l])

    trow = jax.lax.broadcasted_iota(jnp.int32, (_T, 1), 0).astype(jnp.float32)
    w = jnp.where(trow < lenf, 1.0, 0.0)
    agg = jnp.sum(x * w, axis=0, keepdims=True) / (lenf + 1.0)
    emb = jnp.dot(st_ref[0], stw_ref[...], preferred_element_type=jnp.float32)
    emb = emb + stb_ref[...]
    o = jnp.concatenate([agg, emb], axis=1)
    hh = jnp.dot(o, m1w_ref[...], preferred_element_type=jnp.float32) + m1b_ref[...]
    hh = jnp.maximum(hh, 0.0)
    logits = jnp.dot(hh, m2w_ref[...], preferred_element_type=jnp.float32) + m2b_ref[...]
    m = jnp.max(logits, axis=1, keepdims=True)
    e = jnp.exp(logits - m)
    out_ref[0] = e / jnp.sum(e, axis=1, keepdims=True)


def _const_spec(shape):
    nd = len(shape)
    return pl.BlockSpec(shape, lambda *_: (0,) * nd)


@jax.jit
def kernel(X, timestamps, lengths, missing_mask, static, label, R_u, static_W,
           static_b, gnn1_Wv, gnn1_bv, gnn2_Wv, gnn2_bv, tr_Wqkv, tr_bqkv,
           tr_Wo, tr_bo, tr_ln1_g, tr_ln1_b, tr_ln2_g, tr_ln2_b, tr_W1, tr_b1,
           tr_W2, tr_b2, mlp_W1, mlp_b1, mlp_W2, mlp_b2):
    f32 = jnp.float32

    Xt = X.transpose(1, 2, 0).reshape(_B * _F, _T)
    rep = jnp.asarray(np.repeat(np.eye(_T, dtype=np.float32), _D_OB, axis=1))
    Ru2 = R_u.reshape(_F, _D_OB)
    rtile = jnp.broadcast_to(Ru2[None, :, None, :], (_RB // _F, _F, _T, _D_OB))
    rtile = rtile.reshape(_RB, _IN_CH)
    w1t = gnn1_Wv.T
    b1v = gnn1_bv.reshape(1, _IN_CH)

    v1 = pl.pallas_call(
        _gnn1_body,
        grid=(_B * _F // _RB,),
        in_specs=[
            pl.BlockSpec((_RB, _T), lambda i: (i, 0)),
            _const_spec((_T, _IN_CH)),
            _const_spec((_RB, _IN_CH)),
            _const_spec((_IN_CH, _IN_CH)),
            _const_spec((1, _IN_CH)),
        ],
        out_specs=pl.BlockSpec((_RB // _F, _IN_CH), lambda i: (i, 0)),
        out_shape=jax.ShapeDtypeStruct((_B, _IN_CH), f32),
    )(Xt, rep, rtile, w1t, b1v)

    v2 = pl.pallas_call(
        _gnn2_body,
        in_specs=[
            pl.BlockSpec((_B, _IN_CH), lambda: (0, 0)),
            pl.BlockSpec((_IN_CH, _IN_CH), lambda: (0, 0)),
            pl.BlockSpec((1, _IN_CH), lambda: (0, 0)),
        ],
        out_specs=pl.BlockSpec((_B, _IN_CH), lambda: (0, 0)),
        out_shape=jax.ShapeDtypeStruct((_B, _IN_CH), f32),
    )(v1, gnn2_Wv.T, gnn2_bv.reshape(1, _IN_CH))

    u = v2.reshape(_B, _T, _D_OB)

    wq = tr_Wqkv.reshape(_N_LAYERS, 3, _N_HEADS, _DH, _D_TR)
    wq = jnp.pad(wq, ((0, 0), (0, 0), (0, 0), (0, _DHP - _DH), (0, 0)))
    wqkv_p = wq.transpose(0, 4, 1, 2, 3).reshape(_N_LAYERS, _D_TR, 3 * _N_HEADS * _DHP)
    bq = tr_bqkv.reshape(_N_LAYERS, 3, _N_HEADS, _DH)
    bq = jnp.pad(bq, ((0, 0), (0, 0), (0, 0), (0, _DHP - _DH)))
    bqkv_p = bq.reshape(_N_LAYERS, 1, 3 * _N_HEADS * _DHP)
    wo = tr_Wo.transpose(0, 2, 1).reshape(_N_LAYERS, _N_HEADS, _DH, _D_TR)
    wo_p = jnp.pad(wo, ((0, 0), (0, 0), (0, _DHP - _DH), (0, 0)))
    wo_p = wo_p.reshape(_N_LAYERS, _N_HEADS * _DHP, _D_TR)

    tile = jnp.asarray(np.tile(np.eye(_D_OB, dtype=np.float32), (1, _F)))
    itau = jnp.asarray(
        (1.0 / (float(_T) ** np.linspace(0.0, 1.0, _D_PE // 2))).astype(np.float32)
    ).reshape(1, _D_PE // 2)
    tsr = timestamps.T.reshape(_B, _T, 1)
    lenf = lengths.astype(f32).reshape(_B, 1, 1)
    NH3 = 3 * _N_HEADS * _DHP

    def lspec(s2, s3):
        return pl.BlockSpec((1, s2, s3), lambda b: (b, 0, 0))

    def wl(s2, s3):
        return pl.BlockSpec((_N_LAYERS, s2, s3), lambda b: (0, 0, 0))

    probs = pl.pallas_call(
        _tr_body,
        grid=(_B,),
        in_specs=[
            lspec(_T, _D_OB),
            lspec(_T, 1),
            lspec(1, 1),
            lspec(1, _D_STATIC),
            _const_spec((_D_OB, 256)),
            _const_spec((1, _D_PE // 2)),
            wl(_D_TR, NH3),
            wl(1, NH3),
            wl(_N_HEADS * _DHP, _D_TR),
            wl(1, _D_TR),
            wl(1, _D_TR), wl(1, _D_TR),
            wl(1, _D_TR), wl(1, _D_TR),
            wl(_D_TR, _D_INNER),
            wl(1, _D_INNER),
            wl(_D_INNER, _D_TR),
            wl(1, _D_TR),
            _const_spec((_D_STATIC, _F)),
            _const_spec((1, _F)),
            _const_spec((_D_FINAL, _D_FINAL)),
            _const_spec((1, _D_FINAL)),
            _const_spec((_D_FINAL, 2)),
            _const_spec((1, 2)),
        ],
        out_specs=pl.BlockSpec((1, 1, 2), lambda b: (b, 0, 0)),
        out_shape=jax.ShapeDtypeStruct((_B, 1, 2), f32),
    )(
        u, tsr, lenf, static.reshape(_B, 1, _D_STATIC), tile, itau,
        wqkv_p, bqkv_p, wo_p, tr_bo.reshape(_N_LAYERS, 1, _D_TR),
        tr_ln1_g.reshape(_N_LAYERS, 1, _D_TR), tr_ln1_b.reshape(_N_LAYERS, 1, _D_TR),
        tr_ln2_g.reshape(_N_LAYERS, 1, _D_TR), tr_ln2_b.reshape(_N_LAYERS, 1, _D_TR),
        tr_W1.transpose(0, 2, 1), tr_b1.reshape(_N_LAYERS, 1, _D_INNER),
        tr_W2.transpose(0, 2, 1), tr_b2.reshape(_N_LAYERS, 1, _D_TR),
        static_W.T, static_b.reshape(1, _F),
        mlp_W1.T, mlp_b1.reshape(1, _D_FINAL),
        mlp_W2.T, mlp_b2.reshape(1, 2),
    )
    return probs.reshape(_B, 2)


# speedup vs baseline: 7.8505x; 1.1045x over previous
import functools
import math

import jax
import jax.numpy as jnp
import numpy as np
from jax.experimental import pallas as pl
from jax.experimental.pallas import tpu as pltpu

_T = 200
_B = 64
_F = 64
_D_OB = 4
_D_PE = 16
_N_HEADS = 16
_D_INNER = 1024
_N_LAYERS = 2
_D_STATIC = 9
_D_TR = 256 + _D_PE
_IN_CH = _T * _D_OB
_D_FINAL = _D_TR + _F
_DH = _D_TR // _N_HEADS
_DHP = 32
_RB = 512


def _gnn1_body(x_ref, rep_ref, rtile_ref, w1t_ref, b1_ref, out_ref):
    xrep = jnp.dot(x_ref[...], rep_ref[...], preferred_element_type=jnp.float32)
    sd = jnp.maximum(xrep * rtile_ref[...], 0.0)
    z = jnp.dot(sd, w1t_ref[...], preferred_element_type=jnp.float32) + b1_ref[...]
    z = jnp.maximum(z, 0.0)
    z3 = z.reshape(_RB // _F, _F, _IN_CH)
    out_ref[...] = jnp.mean(z3, axis=1)


def _gnn2_body(v1_ref, w2t_ref, b2_ref, out_ref):
    z = jnp.dot(v1_ref[...], w2t_ref[...], preferred_element_type=jnp.float32)
    out_ref[...] = jnp.maximum(z + b2_ref[...], 0.0)


def _ln_in(x, g, b):
    mu = jnp.mean(x, axis=-1, keepdims=True)
    d = x - mu
    v = jnp.mean(d * d, axis=-1, keepdims=True)
    return d * jax.lax.rsqrt(v + 1e-5) * g + b


def _tr_body(u_ref, ts_ref, len_ref, st_ref, tile_ref, itau_ref,
             wqkv_ref, bqkv_ref, wo_ref, bo_ref,
             g1_ref, c1_ref, g2_ref, c2_ref,
             w1_ref, b1_ref, w2_ref, b2_ref,
             stw_ref, stb_ref, m1w_ref, m1b_ref, m2w_ref, m2b_ref,
             out_ref):
    lenf = len_ref[0]
    ts = ts_ref[0]
    scaled = ts * itau_ref[...]
    pe = jnp.concatenate([jnp.sin(scaled), jnp.cos(scaled)], axis=1)
    u = u_ref[0]
    x256 = jnp.dot(u, tile_ref[...], preferred_element_type=jnp.float32)
    x = jnp.concatenate([x256, pe], axis=1)

    col = jax.lax.broadcasted_iota(jnp.int32, (_T, _T), 1).astype(jnp.float32)
    neg = jnp.float32(-1e9)
    scale = jnp.float32(1.0 / math.sqrt(_DH))

    for l in range(_N_LAYERS):
        qkv = jnp.dot(x, wqkv_ref[l], preferred_element_type=jnp.float32)
        qkv = qkv + bqkv_ref[l]
        outs = []
        for h in range(_N_HEADS):
            q = jax.lax.slice(qkv, (0, _DHP * h), (_T, _DHP * h + _DHP))
            k = jax.lax.slice(qkv, (0, 512 + _DHP * h), (_T, 512 + _DHP * h + _DHP))
            v = jax.lax.slice(qkv, (0, 1024 + _DHP * h), (_T, 1024 + _DHP * h + _DHP))
            s = jax.lax.dot_general(q, k, (((1,), (1,)), ((), ())),
                                    preferred_element_type=jnp.float32) * scale
            s = jnp.where(col >= lenf, neg, s)
            m = jnp.max(s, axis=1, keepdims=True)
            e = jnp.exp(s - m)
            r = 1.0 / jnp.sum(e, axis=1, keepdims=True)
            outs.append(jnp.dot(e, v, preferred_element_type=jnp.float32) * r)
        ao = jnp.concatenate(outs, axis=1)
        y = jnp.dot(ao, wo_ref[l], preferred_element_type=jnp.float32)
        x = _ln_in(y + bo_ref[l] + x, g1_ref[l], c1_ref[l])
        ff = jnp.dot(x, w1_ref[l], preferred_element_type=jnp.float32) + b1_ref[l]
        ff = jnp.maximum(ff, 0.0)
        ff = jnp.dot(ff, w2_ref[l], preferred_element_type=jnp.float32) + b2_ref[l]
        x = _ln_in(x + ff, g2_ref[l], c2_ref[l])

    trow = jax.lax.broadcasted_iota(jnp.int32, (_T, 1), 0).astype(jnp.float32)
    w = jnp.where(trow < lenf, 1.0, 0.0)
    agg = jnp.sum(x * w, axis=0, keepdims=True) / (lenf + 1.0)
    emb = jnp.dot(st_ref[0], stw_ref[...], preferred_element_type=jnp.float32)
    emb = emb + stb_ref[...]
    o = jnp.concatenate([agg, emb], axis=1)
    hh = jnp.dot(o, m1w_ref[...], preferred_element_type=jnp.float32) + m1b_ref[...]
    hh = jnp.maximum(hh, 0.0)
    logits = jnp.dot(hh, m2w_ref[...], preferred_element_type=jnp.float32) + m2b_ref[...]
    m = jnp.max(logits, axis=1, keepdims=True)
    e = jnp.exp(logits - m)
    out_ref[0] = e / jnp.sum(e, axis=1, keepdims=True)


def _const_spec(shape):
    nd = len(shape)
    return pl.BlockSpec(shape, lambda *_: (0,) * nd)


@jax.jit
def kernel(X, timestamps, lengths, missing_mask, static, label, R_u, static_W,
           static_b, gnn1_Wv, gnn1_bv, gnn2_Wv, gnn2_bv, tr_Wqkv, tr_bqkv,
           tr_Wo, tr_bo, tr_ln1_g, tr_ln1_b, tr_ln2_g, tr_ln2_b, tr_W1, tr_b1,
           tr_W2, tr_b2, mlp_W1, mlp_b1, mlp_W2, mlp_b2):
    f32 = jnp.float32

    Xt = X.transpose(1, 2, 0).reshape(_B * _F, _T)
    rep = jnp.asarray(np.repeat(np.eye(_T, dtype=np.float32), _D_OB, axis=1))
    Ru2 = R_u.reshape(_F, _D_OB)
    rtile = jnp.broadcast_to(Ru2[None, :, None, :], (_RB // _F, _F, _T, _D_OB))
    rtile = rtile.reshape(_RB, _IN_CH)
    w1t = gnn1_Wv.T
    b1v = gnn1_bv.reshape(1, _IN_CH)

    v1 = pl.pallas_call(
        _gnn1_body,
        grid=(_B * _F // _RB,),
        in_specs=[
            pl.BlockSpec((_RB, _T), lambda i: (i, 0)),
            _const_spec((_T, _IN_CH)),
            _const_spec((_RB, _IN_CH)),
            _const_spec((_IN_CH, _IN_CH)),
            _const_spec((1, _IN_CH)),
        ],
        out_specs=pl.BlockSpec((_RB // _F, _IN_CH), lambda i: (i, 0)),
        out_shape=jax.ShapeDtypeStruct((_B, _IN_CH), f32),
        compiler_params=pltpu.CompilerParams(
            dimension_semantics=("parallel",)),
    )(Xt, rep, rtile, w1t, b1v)

    v2 = pl.pallas_call(
        _gnn2_body,
        in_specs=[
            pl.BlockSpec((_B, _IN_CH), lambda: (0, 0)),
            pl.BlockSpec((_IN_CH, _IN_CH), lambda: (0, 0)),
            pl.BlockSpec((1, _IN_CH), lambda: (0, 0)),
        ],
        out_specs=pl.BlockSpec((_B, _IN_CH), lambda: (0, 0)),
        out_shape=jax.ShapeDtypeStruct((_B, _IN_CH), f32),
    )(v1, gnn2_Wv.T, gnn2_bv.reshape(1, _IN_CH))

    u = v2.reshape(_B, _T, _D_OB)

    wq = tr_Wqkv.reshape(_N_LAYERS, 3, _N_HEADS, _DH, _D_TR)
    wq = jnp.pad(wq, ((0, 0), (0, 0), (0, 0), (0, _DHP - _DH), (0, 0)))
    wqkv_p = wq.transpose(0, 4, 1, 2, 3).reshape(_N_LAYERS, _D_TR, 3 * _N_HEADS * _DHP)
    bq = tr_bqkv.reshape(_N_LAYERS, 3, _N_HEADS, _DH)
    bq = jnp.pad(bq, ((0, 0), (0, 0), (0, 0), (0, _DHP - _DH)))
    bqkv_p = bq.reshape(_N_LAYERS, 1, 3 * _N_HEADS * _DHP)
    wo = tr_Wo.transpose(0, 2, 1).reshape(_N_LAYERS, _N_HEADS, _DH, _D_TR)
    wo_p = jnp.pad(wo, ((0, 0), (0, 0), (0, _DHP - _DH), (0, 0)))
    wo_p = wo_p.reshape(_N_LAYERS, _N_HEADS * _DHP, _D_TR)

    tile = jnp.asarray(np.tile(np.eye(_D_OB, dtype=np.float32), (1, _F)))
    itau = jnp.asarray(
        (1.0 / (float(_T) ** np.linspace(0.0, 1.0, _D_PE // 2))).astype(np.float32)
    ).reshape(1, _D_PE // 2)
    tsr = timestamps.T.reshape(_B, _T, 1)
    lenf = lengths.astype(f32).reshape(_B, 1, 1)
    NH3 = 3 * _N_HEADS * _DHP

    def lspec(s2, s3):
        return pl.BlockSpec((1, s2, s3), lambda b: (b, 0, 0))

    def wl(s2, s3):
        return pl.BlockSpec((_N_LAYERS, s2, s3), lambda b: (0, 0, 0))

    probs = pl.pallas_call(
        _tr_body,
        grid=(_B,),
        in_specs=[
            lspec(_T, _D_OB),
            lspec(_T, 1),
            lspec(1, 1),
            lspec(1, _D_STATIC),
            _const_spec((_D_OB, 256)),
            _const_spec((1, _D_PE // 2)),
            wl(_D_TR, NH3),
            wl(1, NH3),
            wl(_N_HEADS * _DHP, _D_TR),
            wl(1, _D_TR),
            wl(1, _D_TR), wl(1, _D_TR),
            wl(1, _D_TR), wl(1, _D_TR),
            wl(_D_TR, _D_INNER),
            wl(1, _D_INNER),
            wl(_D_INNER, _D_TR),
            wl(1, _D_TR),
            _const_spec((_D_STATIC, _F)),
            _const_spec((1, _F)),
            _const_spec((_D_FINAL, _D_FINAL)),
            _const_spec((1, _D_FINAL)),
            _const_spec((_D_FINAL, 2)),
            _const_spec((1, 2)),
        ],
        out_specs=pl.BlockSpec((1, 1, 2), lambda b: (b, 0, 0)),
        out_shape=jax.ShapeDtypeStruct((_B, 1, 2), f32),
        compiler_params=pltpu.CompilerParams(
            dimension_semantics=("parallel",)),
    )(
        u, tsr, lenf, static.reshape(_B, 1, _D_STATIC), tile, itau,
        wqkv_p, bqkv_p, wo_p, tr_bo.reshape(_N_LAYERS, 1, _D_TR),
        tr_ln1_g.reshape(_N_LAYERS, 1, _D_TR), tr_ln1_b.reshape(_N_LAYERS, 1, _D_TR),
        tr_ln2_g.reshape(_N_LAYERS, 1, _D_TR), tr_ln2_b.reshape(_N_LAYERS, 1, _D_TR),
        tr_W1.transpose(0, 2, 1), tr_b1.reshape(_N_LAYERS, 1, _D_INNER),
        tr_W2.transpose(0, 2, 1), tr_b2.reshape(_N_LAYERS, 1, _D_TR),
        static_W.T, static_b.reshape(1, _F),
        mlp_W1.T, mlp_b1.reshape(1, _D_FINAL),
        mlp_W2.T, mlp_b2.reshape(1, 2),
    )
    return probs.reshape(_B, 2)


# speedup vs baseline: 8.8597x; 1.1286x over previous
import functools
import math

import jax
import jax.numpy as jnp
import numpy as np
from jax.experimental import pallas as pl
from jax.experimental.pallas import tpu as pltpu

_T = 200
_B = 64
_F = 64
_D_OB = 4
_D_PE = 16
_N_HEADS = 16
_D_INNER = 1024
_N_LAYERS = 2
_D_STATIC = 9
_D_TR = 256 + _D_PE
_IN_CH = _T * _D_OB
_D_FINAL = _D_TR + _F
_DH = _D_TR // _N_HEADS
_DHP = 32
_RB = 512


def _gnn1_body(x_ref, rep_ref, rtile_ref, w1t_ref, b1_ref, out_ref):
    xrep = jnp.dot(x_ref[...], rep_ref[...], preferred_element_type=jnp.float32)
    sd = jnp.maximum(xrep * rtile_ref[...], 0.0)
    z = jnp.dot(sd, w1t_ref[...], preferred_element_type=jnp.float32) + b1_ref[...]
    z = jnp.maximum(z, 0.0)
    z3 = z.reshape(_RB // _F, _F, _IN_CH)
    out_ref[...] = jnp.mean(z3, axis=1)


def _gnn2_body(v1_ref, w2t_ref, b2_ref, out_ref):
    z = jnp.dot(v1_ref[...], w2t_ref[...], preferred_element_type=jnp.float32)
    out_ref[...] = jnp.maximum(z + b2_ref[...], 0.0)


def _ln_in(x, g, b):
    mu = jnp.mean(x, axis=-1, keepdims=True)
    d = x - mu
    v = jnp.mean(d * d, axis=-1, keepdims=True)
    return d * jax.lax.rsqrt(v + 1e-5) * g + b


def _tr_body(u_ref, ts_ref, len_ref, st_ref, tile_ref, itau_ref,
             wqkv_ref, bqkv_ref, wo_ref, bo_ref,
             g1_ref, c1_ref, g2_ref, c2_ref,
             w1_ref, b1_ref, w2_ref, b2_ref,
             stw_ref, stb_ref, m1w_ref, m1b_ref, m2w_ref, m2b_ref,
             out_ref):
    lenf = len_ref[0]
    ts = ts_ref[0]
    scaled = ts * itau_ref[...]
    pe = jnp.concatenate([jnp.sin(scaled), jnp.cos(scaled)], axis=1)
    u = u_ref[0]
    x256 = jnp.dot(u, tile_ref[...], preferred_element_type=jnp.float32)
    x = jnp.concatenate([x256, pe], axis=1)

    trow = jax.lax.broadcasted_iota(jnp.int32, (_T, 1), 0).astype(jnp.float32)
    wvec = jnp.where(trow < lenf, 1.0, 0.0)

    for l in range(_N_LAYERS):
        qkv = jnp.dot(x, wqkv_ref[l], preferred_element_type=jnp.float32)
        qkv = qkv + bqkv_ref[l]
        vm = jax.lax.slice(qkv, (0, 1024), (_T, 1536)) * wvec
        outs = []
        for h in range(_N_HEADS):
            q = jax.lax.slice(qkv, (0, _DHP * h), (_T, _DHP * h + _DHP))
            k = jax.lax.slice(qkv, (0, 512 + _DHP * h), (_T, 512 + _DHP * h + _DHP))
            v = jax.lax.slice(vm, (0, _DHP * h), (_T, _DHP * h + _DHP))
            s = jax.lax.dot_general(q, k, (((1,), (1,)), ((), ())),
                                    preferred_element_type=jnp.float32)
            m = jnp.max(s, axis=1, keepdims=True)
            e = jnp.exp(s - m)
            vplus = jnp.concatenate([v, wvec], axis=1)
            o = jnp.dot(e, vplus, preferred_element_type=jnp.float32)
            outs.append(jax.lax.slice(o, (0, 0), (_T, _DHP))
                        * (1.0 / jax.lax.slice(o, (0, _DHP), (_T, _DHP + 1))))
        ao = jnp.concatenate(outs, axis=1)
        y = jnp.dot(ao, wo_ref[l], preferred_element_type=jnp.float32)
        x = _ln_in(y + bo_ref[l] + x, g1_ref[l], c1_ref[l])
        ff = jnp.dot(x, w1_ref[l], preferred_element_type=jnp.float32) + b1_ref[l]
        ff = jnp.maximum(ff, 0.0)
        ff = jnp.dot(ff, w2_ref[l], preferred_element_type=jnp.float32) + b2_ref[l]
        x = _ln_in(x + ff, g2_ref[l], c2_ref[l])

    agg = jnp.sum(x * wvec, axis=0, keepdims=True) / (lenf + 1.0)
    emb = jnp.dot(st_ref[0], stw_ref[...], preferred_element_type=jnp.float32)
    emb = emb + stb_ref[...]
    o = jnp.concatenate([agg, emb], axis=1)
    hh = jnp.dot(o, m1w_ref[...], preferred_element_type=jnp.float32) + m1b_ref[...]
    hh = jnp.maximum(hh, 0.0)
    logits = jnp.dot(hh, m2w_ref[...], preferred_element_type=jnp.float32) + m2b_ref[...]
    m = jnp.max(logits, axis=1, keepdims=True)
    e = jnp.exp(logits - m)
    out_ref[0] = e / jnp.sum(e, axis=1, keepdims=True)


def _const_spec(shape):
    nd = len(shape)
    return pl.BlockSpec(shape, lambda *_: (0,) * nd)


@jax.jit
def kernel(X, timestamps, lengths, missing_mask, static, label, R_u, static_W,
           static_b, gnn1_Wv, gnn1_bv, gnn2_Wv, gnn2_bv, tr_Wqkv, tr_bqkv,
           tr_Wo, tr_bo, tr_ln1_g, tr_ln1_b, tr_ln2_g, tr_ln2_b, tr_W1, tr_b1,
           tr_W2, tr_b2, mlp_W1, mlp_b1, mlp_W2, mlp_b2):
    f32 = jnp.float32

    Xt = X.transpose(1, 2, 0).reshape(_B * _F, _T)
    rep = jnp.asarray(np.repeat(np.eye(_T, dtype=np.float32), _D_OB, axis=1))
    Ru2 = R_u.reshape(_F, _D_OB)
    rtile = jnp.broadcast_to(Ru2[None, :, None, :], (_RB // _F, _F, _T, _D_OB))
    rtile = rtile.reshape(_RB, _IN_CH)
    w1t = gnn1_Wv.T
    b1v = gnn1_bv.reshape(1, _IN_CH)

    v1 = pl.pallas_call(
        _gnn1_body,
        grid=(_B * _F // _RB,),
        in_specs=[
            pl.BlockSpec((_RB, _T), lambda i: (i, 0)),
            _const_spec((_T, _IN_CH)),
            _const_spec((_RB, _IN_CH)),
            _const_spec((_IN_CH, _IN_CH)),
            _const_spec((1, _IN_CH)),
        ],
        out_specs=pl.BlockSpec((_RB // _F, _IN_CH), lambda i: (i, 0)),
        out_shape=jax.ShapeDtypeStruct((_B, _IN_CH), f32),
        compiler_params=pltpu.CompilerParams(
            dimension_semantics=("parallel",)),
    )(Xt, rep, rtile, w1t, b1v)

    v2 = pl.pallas_call(
        _gnn2_body,
        in_specs=[
            pl.BlockSpec((_B, _IN_CH), lambda: (0, 0)),
            pl.BlockSpec((_IN_CH, _IN_CH), lambda: (0, 0)),
            pl.BlockSpec((1, _IN_CH), lambda: (0, 0)),
        ],
        out_specs=pl.BlockSpec((_B, _IN_CH), lambda: (0, 0)),
        out_shape=jax.ShapeDtypeStruct((_B, _IN_CH), f32),
    )(v1, gnn2_Wv.T, gnn2_bv.reshape(1, _IN_CH))

    u = v2.reshape(_B, _T, _D_OB)

    wq = tr_Wqkv.reshape(_N_LAYERS, 3, _N_HEADS, _DH, _D_TR)
    wq = jnp.pad(wq, ((0, 0), (0, 0), (0, 0), (0, _DHP - _DH), (0, 0)))
    qscale = jnp.float32(1.0 / math.sqrt(_DH))
    wq = wq.at[:, 0].multiply(qscale)
    wqkv_p = wq.transpose(0, 4, 1, 2, 3).reshape(_N_LAYERS, _D_TR, 3 * _N_HEADS * _DHP)
    bq = tr_bqkv.reshape(_N_LAYERS, 3, _N_HEADS, _DH)
    bq = jnp.pad(bq, ((0, 0), (0, 0), (0, 0), (0, _DHP - _DH)))
    bq = bq.at[:, 0].multiply(qscale)
    bqkv_p = bq.reshape(_N_LAYERS, 1, 3 * _N_HEADS * _DHP)
    wo = tr_Wo.transpose(0, 2, 1).reshape(_N_LAYERS, _N_HEADS, _DH, _D_TR)
    wo_p = jnp.pad(wo, ((0, 0), (0, 0), (0, _DHP - _DH), (0, 0)))
    wo_p = wo_p.reshape(_N_LAYERS, _N_HEADS * _DHP, _D_TR)

    tile = jnp.asarray(np.tile(np.eye(_D_OB, dtype=np.float32), (1, _F)))
    itau = jnp.asarray(
        (1.0 / (float(_T) ** np.linspace(0.0, 1.0, _D_PE // 2))).astype(np.float32)
    ).reshape(1, _D_PE // 2)
    tsr = timestamps.T.reshape(_B, _T, 1)
    lenf = lengths.astype(f32).reshape(_B, 1, 1)
    NH3 = 3 * _N_HEADS * _DHP

    def lspec(s2, s3):
        return pl.BlockSpec((1, s2, s3), lambda b: (b, 0, 0))

    def wl(s2, s3):
        return pl.BlockSpec((_N_LAYERS, s2, s3), lambda b: (0, 0, 0))

    probs = pl.pallas_call(
        _tr_body,
        grid=(_B,),
        in_specs=[
            lspec(_T, _D_OB),
            lspec(_T, 1),
            lspec(1, 1),
            lspec(1, _D_STATIC),
            _const_spec((_D_OB, 256)),
            _const_spec((1, _D_PE // 2)),
            wl(_D_TR, NH3),
            wl(1, NH3),
            wl(_N_HEADS * _DHP, _D_TR),
            wl(1, _D_TR),
            wl(1, _D_TR), wl(1, _D_TR),
            wl(1, _D_TR), wl(1, _D_TR),
            wl(_D_TR, _D_INNER),
            wl(1, _D_INNER),
            wl(_D_INNER, _D_TR),
            wl(1, _D_TR),
            _const_spec((_D_STATIC, _F)),
            _const_spec((1, _F)),
            _const_spec((_D_FINAL, _D_FINAL)),
            _const_spec((1, _D_FINAL)),
            _const_spec((_D_FINAL, 2)),
            _const_spec((1, 2)),
        ],
        out_specs=pl.BlockSpec((1, 1, 2), lambda b: (b, 0, 0)),
        out_shape=jax.ShapeDtypeStruct((_B, 1, 2), f32),
        compiler_params=pltpu.CompilerParams(
            dimension_semantics=("parallel",)),
    )(
        u, tsr, lenf, static.reshape(_B, 1, _D_STATIC), tile, itau,
        wqkv_p, bqkv_p, wo_p, tr_bo.reshape(_N_LAYERS, 1, _D_TR),
        tr_ln1_g.reshape(_N_LAYERS, 1, _D_TR), tr_ln1_b.reshape(_N_LAYERS, 1, _D_TR),
        tr_ln2_g.reshape(_N_LAYERS, 1, _D_TR), tr_ln2_b.reshape(_N_LAYERS, 1, _D_TR),
        tr_W1.transpose(0, 2, 1), tr_b1.reshape(_N_LAYERS, 1, _D_INNER),
        tr_W2.transpose(0, 2, 1), tr_b2.reshape(_N_LAYERS, 1, _D_TR),
        static_W.T, static_b.reshape(1, _F),
        mlp_W1.T, mlp_b1.reshape(1, _D_FINAL),
        mlp_W2.T, mlp_b2.reshape(1, 2),
    )
    return probs.reshape(_B, 2)
